# baseline (speedup 1.0000x reference)
import functools

import jax
import jax.numpy as jnp
from jax import lax
from jax.experimental import pallas as pl
from jax.experimental.pallas import tpu as pltpu
from jax.experimental.pallas import tpu_sc as plsc

N = 10000
E = 320000
D = 128

NC = 2
NS = 16
NW = NC * NS
EPW = E // NW
CH = 80
NCHUNK = EPW // CH
NBUF = 3
NGRP = NCHUNK // NBUF
NTAIL = NCHUNK % NBUF
NP = 10240
RPS = NP // NS
ZROWS = 128
DEGW = 128

_mesh = plsc.VectorSubcoreMesh(core_axis_name="c", subcore_axis_name="s")


@functools.partial(
    pl.kernel,
    out_type=jax.ShapeDtypeStruct((NC, NP, DEGW), jnp.float32),
    mesh=_mesh,
    scratch_types=[
        pltpu.VMEM((NCHUNK, CH), jnp.int32),
        pltpu.VMEM((CH, DEGW), jnp.float32),
        pltpu.VMEM_SHARED((NP, DEGW), jnp.float32),
        pltpu.SemaphoreType.DMA,
        pltpu.SemaphoreType.DMA,
        pltpu.SemaphoreType.DMA,
        pltpu.SemaphoreType.DMA,
        pltpu.SemaphoreType.DMA,
    ],
)
def _deg_sc(dst_hbm, ones_hbm, out_hbm, dst_v, ones_v, acc_sh,
            s0, s1, s2, s3, s4):
    c = lax.axis_index("c")
    s = lax.axis_index("s")
    w = s * NC + c
    sems = (s0, s1, s2, s3, s4)

    pltpu.sync_copy(dst_hbm.at[w], dst_v)

    zeros16 = jnp.zeros((16,), jnp.float32)

    def zfill(i, carry):
        for j in range(DEGW // 16):
            ones_v[i, pl.ds(j * 16, 16)] = zeros16
        return carry

    lax.fori_loop(0, CH, zfill, None)
    base = s * RPS
    for k in range(RPS // CH):
        pltpu.sync_copy(ones_v, acc_sh.at[pl.ds(base + k * CH, CH)])
    pltpu.sync_copy(ones_hbm, ones_v)
    plsc.subcore_barrier()

    def grp(g, carry):
        c0 = g * 5
        cps = [
            pltpu.async_copy(ones_v, acc_sh.at[dst_v.at[c0 + b]], sems[b], add=True)
            for b in range(5)
        ]
        for cp in cps:
            cp.wait()
        return carry

    lax.fori_loop(0, NCHUNK // 5, grp, None)

    plsc.subcore_barrier()
    pltpu.sync_copy(acc_sh.at[pl.ds(base, RPS)], out_hbm.at[c, pl.ds(base, RPS)])


@functools.partial(
    pl.kernel,
    out_type=jax.ShapeDtypeStruct((NC, NP, D), jnp.float32),
    mesh=_mesh,
    scratch_types=[
        pltpu.VMEM((NBUF, CH), jnp.int32),
        pltpu.VMEM((2, NBUF, CH), jnp.int32),
        pltpu.VMEM((NBUF, CH, D), jnp.float32),
        pltpu.VMEM_SHARED((NP, D), jnp.float32),
        pltpu.SemaphoreType.DMA,
        pltpu.SemaphoreType.DMA,
        pltpu.SemaphoreType.DMA,
        pltpu.SemaphoreType.DMA,
        pltpu.SemaphoreType.DMA,
        pltpu.SemaphoreType.DMA,
        pltpu.SemaphoreType.DMA,
        pltpu.SemaphoreType.DMA,
        pltpu.SemaphoreType.DMA,
    ],
)
def _scat_sc(y_hbm, src_hbm, dst_hbm, out_hbm, src_v, dst_v, rows_v,
             acc_sh, i0, i1, i2, g0, g1, g2, s0, s1, s2):
    c = lax.axis_index("c")
    s = lax.axis_index("s")
    w = s * NC + c
    isems = (i0, i1, i2)
    gsems = (g0, g1, g2)
    ssems = (s0, s1, s2)

    zeros16 = jnp.zeros((16,), jnp.float32)

    def zfill(i, carry):
        for j in range(D // 16):
            rows_v[0, i, pl.ds(j * 16, 16)] = zeros16
        return carry

    lax.fori_loop(0, CH, zfill, None)
    base = s * RPS
    for k in range(RPS // CH):
        pltpu.sync_copy(rows_v.at[0], acc_sh.at[pl.ds(base + k * CH, CH)])
    plsc.subcore_barrier()

    def grp(g, carry):
        pr = lax.rem(g, 2)
        c0 = g * NBUF
        icps = []
        for b in range(NBUF):
            e0 = w * EPW + (c0 + b) * CH
            icps.append((
                pltpu.async_copy(src_hbm.at[pl.ds(e0, CH)], src_v.at[b], isems[b]),
                pltpu.async_copy(dst_hbm.at[pl.ds(e0, CH)], dst_v.at[pr, b], isems[b]),
            ))
        gcps = []
        for b in range(NBUF):
            icps[b][0].wait()

            @pl.when(g > 0)
            def _drain_prev(b=b, pr=pr):
                pltpu.make_async_copy(
                    rows_v.at[b], acc_sh.at[dst_v.at[1 - pr, b]], ssems[b]
                ).wait()

            gcps.append(pltpu.async_copy(y_hbm.at[src_v.at[b]], rows_v.at[b], gsems[b]))
        for b in range(NBUF):
            gcps[b].wait()
            icps[b][1].wait()
            pltpu.async_copy(rows_v.at[b], acc_sh.at[dst_v.at[pr, b]], ssems[b],
                             add=True)
        return carry

    lax.fori_loop(0, NGRP, grp, None)

    pr_last = (NGRP - 1) % 2
    for b in range(NBUF):
        pltpu.make_async_copy(
            rows_v.at[b], acc_sh.at[dst_v.at[pr_last, b]], ssems[b]
        ).wait()

    if NTAIL:
        pr_t = NGRP % 2
        icps = []
        for b in range(NTAIL):
            e0 = w * EPW + (NGRP * NBUF + b) * CH
            icps.append((
                pltpu.async_copy(src_hbm.at[pl.ds(e0, CH)], src_v.at[b], isems[b]),
                pltpu.async_copy(dst_hbm.at[pl.ds(e0, CH)], dst_v.at[pr_t, b], isems[b]),
            ))
        gcps = []
        for b in range(NTAIL):
            icps[b][0].wait()
            gcps.append(pltpu.async_copy(y_hbm.at[src_v.at[b]], rows_v.at[b], gsems[b]))
        for b in range(NTAIL):
            gcps[b].wait()
            icps[b][1].wait()
            pltpu.sync_copy(rows_v.at[b], acc_sh.at[dst_v.at[pr_t, b]], add=True)

    plsc.subcore_barrier()
    pltpu.sync_copy(acc_sh.at[pl.ds(base, RPS)], out_hbm.at[c, pl.ds(base, RPS)])


_B = 2000
_G = N // _B

_DOT_DIMS = (((1,), (1,)), ((), ()))


DGB = DEGW


def _dinv_block(dg_ref):
    deg = dg_ref[0] + dg_ref[1] + 1.0
    return lax.rsqrt(deg)[:, 0:1]


def _tc1a_body(x_ref, w_ref, xw_ref):
    xw_ref[...] = lax.dot_general(x_ref[...], w_ref[...], _DOT_DIMS,
                                  preferred_element_type=jnp.float32)


def _tc1a(x, W):
    return pl.pallas_call(
        _tc1a_body,
        grid=(_G,),
        in_specs=[
            pl.BlockSpec((_B, D), lambda i: (i, 0)),
            pl.BlockSpec((D, D), lambda i: (0, 0)),
        ],
        out_specs=pl.BlockSpec((_B, D), lambda i: (i, 0)),
        out_shape=jax.ShapeDtypeStruct((N, D), jnp.float32),
    )(x, W)


def _tc1b_body(xw_ref, dg_ref, y_ref):
    y_ref[...] = xw_ref[...] * _dinv_block(dg_ref)


def _tc1b(xw, degw):
    return pl.pallas_call(
        _tc1b_body,
        grid=(_G,),
        in_specs=[
            pl.BlockSpec((_B, D), lambda i: (i, 0)),
            pl.BlockSpec((NC, _B, DGB), lambda i: (0, i, 0)),
        ],
        out_specs=pl.BlockSpec((_B, D), lambda i: (i, 0)),
        out_shape=jax.ShapeDtypeStruct((N, D), jnp.float32),
    )(xw, degw)


def _tc2_body(y1_ref, p_ref, dg_ref, b_ref, w_ref, y2_ref):
    dinv = _dinv_block(dg_ref)
    h = dinv * (p_ref[0] + p_ref[1] + y1_ref[...]) + b_ref[...]
    xw = lax.dot_general(h, w_ref[...], _DOT_DIMS,
                         preferred_element_type=jnp.float32)
    y2_ref[...] = xw * dinv


def _tc2(y1, p, degw, b1, W2):
    return pl.pallas_call(
        _tc2_body,
        grid=(_G,),
        in_specs=[
            pl.BlockSpec((_B, D), lambda i: (i, 0)),
            pl.BlockSpec((NC, _B, D), lambda i: (0, i, 0)),
            pl.BlockSpec((NC, _B, DGB), lambda i: (0, i, 0)),
            pl.BlockSpec((1, D), lambda i: (0, 0)),
            pl.BlockSpec((D, D), lambda i: (0, 0)),
        ],
        out_specs=pl.BlockSpec((_B, D), lambda i: (i, 0)),
        out_shape=jax.ShapeDtypeStruct((N, D), jnp.float32),
    )(y1, p, degw, b1, W2)


def _tc3_body(y2_ref, q_ref, dg_ref, b_ref, o_ref):
    dinv = _dinv_block(dg_ref)
    o_ref[...] = dinv * (q_ref[0] + q_ref[1] + y2_ref[...]) + b_ref[...]


def _tc3(y2, q, degw, b2):
    return pl.pallas_call(
        _tc3_body,
        grid=(_G,),
        in_specs=[
            pl.BlockSpec((_B, D), lambda i: (i, 0)),
            pl.BlockSpec((NC, _B, D), lambda i: (0, i, 0)),
            pl.BlockSpec((NC, _B, DGB), lambda i: (0, i, 0)),
            pl.BlockSpec((1, D), lambda i: (0, 0)),
        ],
        out_specs=pl.BlockSpec((_B, D), lambda i: (i, 0)),
        out_shape=jax.ShapeDtypeStruct((N, D), jnp.float32),
    )(y2, q, degw, b2)


def kernel(x, edge_index, W1, b1, W2, b2):
    src = edge_index[0]
    dst = edge_index[1]
    dst3 = dst.reshape(NW, NCHUNK, CH)
    ones = jnp.ones((CH, DEGW), jnp.float32)
    degw = _deg_sc(dst3, ones)
    xw1 = _tc1a(x, W1)
    y1 = _tc1b(xw1, degw)
    p = _scat_sc(y1, src, dst)
    y2 = _tc2(y1, p, degw, b1.reshape(1, D), W2)
    q = _scat_sc(y2, src, dst)
    return _tc3(y2, q, degw, b2.reshape(1, D))


# speedup vs baseline: 1.0008x; 1.0008x over previous
import functools

import jax
import jax.numpy as jnp
from jax import lax
from jax.experimental import pallas as pl
from jax.experimental.pallas import tpu as pltpu
from jax.experimental.pallas import tpu_sc as plsc

N = 10000
E = 320000
D = 128

NC = 2
NS = 16
NW = NC * NS
EPW = E // NW
CH = 40
NCHUNK = EPW // CH
NBUF = 6
NGRP = NCHUNK // NBUF
NTAIL = NCHUNK % NBUF
NP = 10240
RPS = NP // NS
ZROWS = 128
DEGW = 128

_mesh = plsc.VectorSubcoreMesh(core_axis_name="c", subcore_axis_name="s")


@functools.partial(
    pl.kernel,
    out_type=jax.ShapeDtypeStruct((NC, NP, DEGW), jnp.float32),
    mesh=_mesh,
    scratch_types=[
        pltpu.VMEM((NCHUNK, CH), jnp.int32),
        pltpu.VMEM((CH, DEGW), jnp.float32),
        pltpu.VMEM_SHARED((NP, DEGW), jnp.float32),
        pltpu.SemaphoreType.DMA,
        pltpu.SemaphoreType.DMA,
        pltpu.SemaphoreType.DMA,
        pltpu.SemaphoreType.DMA,
        pltpu.SemaphoreType.DMA,
    ],
)
def _deg_sc(dst_hbm, ones_hbm, out_hbm, dst_v, ones_v, acc_sh,
            s0, s1, s2, s3, s4):
    c = lax.axis_index("c")
    s = lax.axis_index("s")
    w = s * NC + c
    sems = (s0, s1, s2, s3, s4)

    pltpu.sync_copy(dst_hbm.at[w], dst_v)

    zeros16 = jnp.zeros((16,), jnp.float32)

    def zfill(i, carry):
        for j in range(DEGW // 16):
            ones_v[i, pl.ds(j * 16, 16)] = zeros16
        return carry

    lax.fori_loop(0, CH, zfill, None)
    base = s * RPS
    for k in range(RPS // CH):
        pltpu.sync_copy(ones_v, acc_sh.at[pl.ds(base + k * CH, CH)])
    pltpu.sync_copy(ones_hbm, ones_v)
    plsc.subcore_barrier()

    def grp(g, carry):
        c0 = g * 5
        cps = [
            pltpu.async_copy(ones_v, acc_sh.at[dst_v.at[c0 + b]], sems[b], add=True)
            for b in range(5)
        ]
        for cp in cps:
            cp.wait()
        return carry

    lax.fori_loop(0, NCHUNK // 5, grp, None)

    plsc.subcore_barrier()
    pltpu.sync_copy(acc_sh.at[pl.ds(base, RPS)], out_hbm.at[c, pl.ds(base, RPS)])


@functools.partial(
    pl.kernel,
    out_type=jax.ShapeDtypeStruct((NC, NP, D), jnp.float32),
    mesh=_mesh,
    scratch_types=[
        pltpu.VMEM((NBUF, CH), jnp.int32),
        pltpu.VMEM((2, NBUF, CH), jnp.int32),
        pltpu.VMEM((NBUF, CH, D), jnp.float32),
        pltpu.VMEM_SHARED((NP, D), jnp.float32),
    ] + [pltpu.SemaphoreType.DMA] * (3 * NBUF),
)
def _scat_sc(y_hbm, src_hbm, dst_hbm, out_hbm, src_v, dst_v, rows_v,
             acc_sh, *sems):
    c = lax.axis_index("c")
    s = lax.axis_index("s")
    w = s * NC + c
    isems = sems[:NBUF]
    gsems = sems[NBUF:2 * NBUF]
    ssems = sems[2 * NBUF:]

    zeros16 = jnp.zeros((16,), jnp.float32)

    def zfill(i, carry):
        for j in range(D // 16):
            rows_v[0, i, pl.ds(j * 16, 16)] = zeros16
        return carry

    lax.fori_loop(0, CH, zfill, None)
    base = s * RPS
    for k in range(RPS // CH):
        pltpu.sync_copy(rows_v.at[0], acc_sh.at[pl.ds(base + k * CH, CH)])
    plsc.subcore_barrier()

    def grp(g, carry):
        pr = lax.rem(g, 2)
        c0 = g * NBUF
        icps = []
        for b in range(NBUF):
            e0 = w * EPW + (c0 + b) * CH
            icps.append((
                pltpu.async_copy(src_hbm.at[pl.ds(e0, CH)], src_v.at[b], isems[b]),
                pltpu.async_copy(dst_hbm.at[pl.ds(e0, CH)], dst_v.at[pr, b], isems[b]),
            ))
        gcps = []
        for b in range(NBUF):
            icps[b][0].wait()

            @pl.when(g > 0)
            def _drain_prev(b=b, pr=pr):
                pltpu.make_async_copy(
                    rows_v.at[b], acc_sh.at[dst_v.at[1 - pr, b]], ssems[b]
                ).wait()

            gcps.append(pltpu.async_copy(y_hbm.at[src_v.at[b]], rows_v.at[b], gsems[b]))
        for b in range(NBUF):
            gcps[b].wait()
            icps[b][1].wait()
            pltpu.async_copy(rows_v.at[b], acc_sh.at[dst_v.at[pr, b]], ssems[b],
                             add=True)
        return carry

    lax.fori_loop(0, NGRP, grp, None)

    pr_last = (NGRP - 1) % 2
    for b in range(NBUF):
        pltpu.make_async_copy(
            rows_v.at[b], acc_sh.at[dst_v.at[pr_last, b]], ssems[b]
        ).wait()

    if NTAIL:
        pr_t = NGRP % 2
        icps = []
        for b in range(NTAIL):
            e0 = w * EPW + (NGRP * NBUF + b) * CH
            icps.append((
                pltpu.async_copy(src_hbm.at[pl.ds(e0, CH)], src_v.at[b], isems[b]),
                pltpu.async_copy(dst_hbm.at[pl.ds(e0, CH)], dst_v.at[pr_t, b], isems[b]),
            ))
        gcps = []
        for b in range(NTAIL):
            icps[b][0].wait()
            gcps.append(pltpu.async_copy(y_hbm.at[src_v.at[b]], rows_v.at[b], gsems[b]))
        for b in range(NTAIL):
            gcps[b].wait()
            icps[b][1].wait()
            pltpu.sync_copy(rows_v.at[b], acc_sh.at[dst_v.at[pr_t, b]], add=True)

    plsc.subcore_barrier()
    pltpu.sync_copy(acc_sh.at[pl.ds(base, RPS)], out_hbm.at[c, pl.ds(base, RPS)])


_B = 2000
_G = N // _B

_DOT_DIMS = (((1,), (1,)), ((), ()))


DGB = DEGW


def _dinv_block(dg_ref):
    deg = dg_ref[0] + dg_ref[1] + 1.0
    return lax.rsqrt(deg)[:, 0:1]


def _tc1a_body(x_ref, w_ref, xw_ref):
    xw_ref[...] = lax.dot_general(x_ref[...], w_ref[...], _DOT_DIMS,
                                  preferred_element_type=jnp.float32)


def _tc1a(x, W):
    return pl.pallas_call(
        _tc1a_body,
        grid=(_G,),
        in_specs=[
            pl.BlockSpec((_B, D), lambda i: (i, 0)),
            pl.BlockSpec((D, D), lambda i: (0, 0)),
        ],
        out_specs=pl.BlockSpec((_B, D), lambda i: (i, 0)),
        out_shape=jax.ShapeDtypeStruct((N, D), jnp.float32),
    )(x, W)


def _tc1b_body(xw_ref, dg_ref, y_ref):
    y_ref[...] = xw_ref[...] * _dinv_block(dg_ref)


def _tc1b(xw, degw):
    return pl.pallas_call(
        _tc1b_body,
        grid=(_G,),
        in_specs=[
            pl.BlockSpec((_B, D), lambda i: (i, 0)),
            pl.BlockSpec((NC, _B, DGB), lambda i: (0, i, 0)),
        ],
        out_specs=pl.BlockSpec((_B, D), lambda i: (i, 0)),
        out_shape=jax.ShapeDtypeStruct((N, D), jnp.float32),
    )(xw, degw)


def _tc2_body(y1_ref, p_ref, dg_ref, b_ref, w_ref, y2_ref):
    dinv = _dinv_block(dg_ref)
    h = dinv * (p_ref[0] + p_ref[1] + y1_ref[...]) + b_ref[...]
    xw = lax.dot_general(h, w_ref[...], _DOT_DIMS,
                         preferred_element_type=jnp.float32)
    y2_ref[...] = xw * dinv


def _tc2(y1, p, degw, b1, W2):
    return pl.pallas_call(
        _tc2_body,
        grid=(_G,),
        in_specs=[
            pl.BlockSpec((_B, D), lambda i: (i, 0)),
            pl.BlockSpec((NC, _B, D), lambda i: (0, i, 0)),
            pl.BlockSpec((NC, _B, DGB), lambda i: (0, i, 0)),
            pl.BlockSpec((1, D), lambda i: (0, 0)),
            pl.BlockSpec((D, D), lambda i: (0, 0)),
        ],
        out_specs=pl.BlockSpec((_B, D), lambda i: (i, 0)),
        out_shape=jax.ShapeDtypeStruct((N, D), jnp.float32),
    )(y1, p, degw, b1, W2)


def _tc3_body(y2_ref, q_ref, dg_ref, b_ref, o_ref):
    dinv = _dinv_block(dg_ref)
    o_ref[...] = dinv * (q_ref[0] + q_ref[1] + y2_ref[...]) + b_ref[...]


def _tc3(y2, q, degw, b2):
    return pl.pallas_call(
        _tc3_body,
        grid=(_G,),
        in_specs=[
            pl.BlockSpec((_B, D), lambda i: (i, 0)),
            pl.BlockSpec((NC, _B, D), lambda i: (0, i, 0)),
            pl.BlockSpec((NC, _B, DGB), lambda i: (0, i, 0)),
            pl.BlockSpec((1, D), lambda i: (0, 0)),
        ],
        out_specs=pl.BlockSpec((_B, D), lambda i: (i, 0)),
        out_shape=jax.ShapeDtypeStruct((N, D), jnp.float32),
    )(y2, q, degw, b2)


def kernel(x, edge_index, W1, b1, W2, b2):
    src = edge_index[0]
    dst = edge_index[1]
    dst3 = dst.reshape(NW, NCHUNK, CH)
    ones = jnp.ones((CH, DEGW), jnp.float32)
    degw = _deg_sc(dst3, ones)
    y1 = _tc1b(_tc1a(x, W1), degw)
    p = _scat_sc(y1, src, dst)
    y2 = _tc2(y1, p, degw, b1.reshape(1, D), W2)
    q = _scat_sc(y2, src, dst)
    return _tc3(y2, q, degw, b2.reshape(1, D))
